# Initial kernel scaffold; baseline (speedup 1.0000x reference)
import jax
import jax.numpy as jnp
from jax.experimental import pallas as pl


def kernel(x, edge_index, edge_type, basis1, comp1, root1, bias1, basis2, comp2, root2, bias2):
    raise NotImplementedError("write your pallas kernel here")


# speedup vs baseline: 8.0940x; 8.0940x over previous
import functools

import jax
import jax.numpy as jnp
from jax import lax
from jax.experimental import pallas as pl
from jax.experimental.pallas import tpu as pltpu
from jax.experimental.pallas import tpu_sc as plsc

N = 10000
E = 320000
IN_C = 128
HID = 16
R = 40
NC = 16
NR = N * R
NR_PAD = 409600

NCORES = 2
NSUB = 16
NW = NCORES * NSUB
EW = E // NW
BT = 80
NBATCH = EW // BT

N_PAD = 10240
ROWS_T = N_PAD // NSUB
CNT_T = NR_PAD // NSUB
CNT_CH = 3200
NB_ROWS = 2000
GRID_N = N // NB_ROWS


def _sc_mesh():
    return plsc.VectorSubcoreMesh(core_axis_name="c", subcore_axis_name="s")


def _fill1d(ref, n, val):
    @pl.loop(0, n // 16)
    def _(i):
        ref[pl.ds(i * 16, 16)] = jnp.full((16,), val, ref.dtype)


def _fill2d(ref, rows, val):
    @pl.loop(0, rows)
    def _(i):
        ref[i, :] = jnp.full((16,), val, ref.dtype)


@functools.partial(
    pl.kernel,
    out_type=jax.ShapeDtypeStruct((NCORES * NR_PAD,), jnp.float32),
    mesh=_sc_mesh(),
    scratch_types=[
        pltpu.VMEM((BT,), jnp.int32),
        pltpu.VMEM((BT,), jnp.float32),
        pltpu.VMEM((CNT_CH,), jnp.float32),
        pltpu.VMEM_SHARED((NR_PAD,), jnp.float32),
    ],
)
def _sc_count(cidx_hbm, out_hbm, cidx_v, ones_v, buf_v, cnt_sh):
    c = lax.axis_index("c")
    s = lax.axis_index("s")
    wid = s * NCORES + c

    _fill1d(buf_v, CNT_CH, 0.0)

    @pl.loop(0, CNT_T // CNT_CH)
    def _(k):
        off = pl.multiple_of(s * CNT_T + k * CNT_CH, 8)
        pltpu.sync_copy(buf_v, cnt_sh.at[pl.ds(off, CNT_CH)])

    _fill1d(ones_v, BT, 1.0)
    plsc.subcore_barrier()

    @pl.loop(0, NBATCH)
    def _(j):
        base = pl.multiple_of(wid * EW + j * BT, 8)
        pltpu.sync_copy(cidx_hbm.at[pl.ds(base, BT)], cidx_v)
        pltpu.sync_copy(ones_v, cnt_sh.at[cidx_v], add=True)

    plsc.subcore_barrier()

    @pl.loop(0, CNT_T // CNT_CH)
    def _(k):
        off = pl.multiple_of(s * CNT_T + k * CNT_CH, 8)
        pltpu.sync_copy(cnt_sh.at[pl.ds(off, CNT_CH)], buf_v)
        dst_off = pl.multiple_of(c * NR_PAD + s * CNT_T + k * CNT_CH, 8)
        pltpu.sync_copy(buf_v, out_hbm.at[pl.ds(dst_off, CNT_CH)])


@functools.partial(
    pl.kernel,
    out_type=jax.ShapeDtypeStruct((E,), jnp.float32),
    mesh=_sc_mesh(),
    scratch_types=[
        pltpu.VMEM((BT,), jnp.int32),
        pltpu.VMEM((BT,), jnp.float32),
        pltpu.SemaphoreType.DMA,
    ],
)
def _sc_norm(cidx_hbm, inv_hbm, out_hbm, cidx_v, nv, sem):
    c = lax.axis_index("c")
    s = lax.axis_index("s")
    wid = s * NCORES + c

    @pl.loop(0, NBATCH)
    def _(j):
        base = pl.multiple_of(wid * EW + j * BT, 8)
        pltpu.sync_copy(cidx_hbm.at[pl.ds(base, BT)], cidx_v)
        pltpu.async_copy(inv_hbm.at[cidx_v], nv, sem).wait()
        pltpu.sync_copy(nv, out_hbm.at[pl.ds(base, BT)])


@functools.partial(
    pl.kernel,
    out_type=jax.ShapeDtypeStruct((NCORES * N_PAD, 16), jnp.float32),
    mesh=_sc_mesh(),
    compiler_params=pltpu.CompilerParams(use_tc_tiling_on_sc=False),
    scratch_types=[
        pltpu.VMEM((BT,), jnp.int32),
        pltpu.VMEM((BT,), jnp.int32),
        pltpu.VMEM((BT,), jnp.float32),
        pltpu.VMEM((BT, 16), jnp.float32),
        pltpu.VMEM((ROWS_T, 16), jnp.float32),
        pltpu.VMEM_SHARED((N_PAD, 16), jnp.float32),
        pltpu.SemaphoreType.DMA,
    ],
)
def _sc_msg(gidx_hbm, dst_hbm, norm_hbm, h_hbm, out_hbm,
            gidx_v, dst_v, norm_v, rows_v, tile_v, agg_sh, sem):
    c = lax.axis_index("c")
    s = lax.axis_index("s")
    wid = s * NCORES + c

    _fill2d(tile_v, ROWS_T, 0.0)
    row0 = pl.multiple_of(s * ROWS_T, 8)
    pltpu.sync_copy(tile_v, agg_sh.at[pl.ds(row0, ROWS_T), :])
    plsc.subcore_barrier()

    @pl.loop(0, NBATCH)
    def _(j):
        base = pl.multiple_of(wid * EW + j * BT, 8)
        pltpu.sync_copy(gidx_hbm.at[pl.ds(base, BT)], gidx_v)
        pltpu.sync_copy(dst_hbm.at[pl.ds(base, BT)], dst_v)
        pltpu.sync_copy(norm_hbm.at[pl.ds(base, BT)], norm_v)
        pltpu.async_copy(h_hbm.at[gidx_v], rows_v, sem).wait()

        @pl.loop(0, BT // 16)
        def _(g):
            nv = norm_v[pl.ds(g * 16, 16)]
            for k in range(16):
                e = g * 16 + k
                rows_v[e, :] = rows_v[e, :] * nv[k]

        pltpu.sync_copy(rows_v, agg_sh.at[dst_v], add=True)

    plsc.subcore_barrier()
    pltpu.sync_copy(agg_sh.at[pl.ds(row0, ROWS_T), :], tile_v)
    out_row = pl.multiple_of(c * N_PAD + s * ROWS_T, 8)
    pltpu.sync_copy(tile_v, out_hbm.at[pl.ds(out_row, ROWS_T), :])


def _edge_body(s_ref, d_ref, t_ref, g_ref, c_ref):
    t = t_ref[...]
    g_ref[...] = s_ref[...] * R + t
    c_ref[...] = d_ref[...] * R + t


def _tc_edge(src2, dst2, et2):
    return pl.pallas_call(
        _edge_body,
        out_shape=(
            jax.ShapeDtypeStruct(src2.shape, jnp.int32),
            jax.ShapeDtypeStruct(src2.shape, jnp.int32),
        ),
    )(src2, dst2, et2)


def _inv_body(c_ref, o_ref):
    o_ref[...] = 1.0 / jnp.maximum(c_ref[0] + c_ref[1], 1.0)


def _tc_inv(counts3):
    return pl.pallas_call(
        _inv_body,
        out_shape=jax.ShapeDtypeStruct(counts3.shape[1:], jnp.float32),
    )(counts3)


def _prep_body(x_ref, wf_ref, r_ref, b_ref, h_ref, xr_ref):
    xv = x_ref[...]
    h_ref[...] = jnp.dot(xv, wf_ref[...], preferred_element_type=jnp.float32)
    xr_ref[...] = (
        jnp.dot(xv, r_ref[...], preferred_element_type=jnp.float32) + b_ref[...]
    )


def _tc_prep(x, wflat, root, bias2d):
    k = x.shape[1]
    m = wflat.shape[1]
    return pl.pallas_call(
        _prep_body,
        grid=(GRID_N,),
        in_specs=[
            pl.BlockSpec((NB_ROWS, k), lambda i: (i, 0)),
            pl.BlockSpec((k, m), lambda i: (0, 0)),
            pl.BlockSpec((k, 16), lambda i: (0, 0)),
            pl.BlockSpec((1, 16), lambda i: (0, 0)),
        ],
        out_specs=(
            pl.BlockSpec((NB_ROWS, m), lambda i: (i, 0)),
            pl.BlockSpec((NB_ROWS, 16), lambda i: (i, 0)),
        ),
        out_shape=(
            jax.ShapeDtypeStruct((N, m), jnp.float32),
            jax.ShapeDtypeStruct((N, 16), jnp.float32),
        ),
    )(x, wflat, root, bias2d)


def _mid_body(agg_ref, xr_ref, wf_ref, r_ref, b_ref, h2_ref, hr_ref):
    h = jnp.maximum(agg_ref[0] + agg_ref[1] + xr_ref[...], 0.0)
    h2_ref[...] = jnp.dot(h, wf_ref[...], preferred_element_type=jnp.float32)
    hr_ref[...] = (
        jnp.dot(h, r_ref[...], preferred_element_type=jnp.float32) + b_ref[...]
    )


def _tc_mid(agg3, xr, wflat2, root2, bias2d):
    m = wflat2.shape[1]
    return pl.pallas_call(
        _mid_body,
        grid=(GRID_N,),
        in_specs=[
            pl.BlockSpec((2, NB_ROWS, 16), lambda i: (0, i, 0)),
            pl.BlockSpec((NB_ROWS, 16), lambda i: (i, 0)),
            pl.BlockSpec((16, m), lambda i: (0, 0)),
            pl.BlockSpec((16, 16), lambda i: (0, 0)),
            pl.BlockSpec((1, 16), lambda i: (0, 0)),
        ],
        out_specs=(
            pl.BlockSpec((NB_ROWS, m), lambda i: (i, 0)),
            pl.BlockSpec((NB_ROWS, 16), lambda i: (i, 0)),
        ),
        out_shape=(
            jax.ShapeDtypeStruct((N, m), jnp.float32),
            jax.ShapeDtypeStruct((N, 16), jnp.float32),
        ),
    )(agg3, xr, wflat2, root2, bias2d)


def _fin_body(agg_ref, hr_ref, o_ref):
    v = agg_ref[0] + agg_ref[1] + hr_ref[...]
    m = jnp.max(v, axis=1, keepdims=True)
    ex = jnp.exp(v - m)
    o_ref[...] = (v - m) - jnp.log(jnp.sum(ex, axis=1, keepdims=True))


def _tc_fin(agg3, hr):
    return pl.pallas_call(
        _fin_body,
        grid=(GRID_N,),
        in_specs=[
            pl.BlockSpec((2, NB_ROWS, 16), lambda i: (0, i, 0)),
            pl.BlockSpec((NB_ROWS, 16), lambda i: (i, 0)),
        ],
        out_specs=pl.BlockSpec((NB_ROWS, 16), lambda i: (i, 0)),
        out_shape=jax.ShapeDtypeStruct((N, 16), jnp.float32),
    )(agg3, hr)


def kernel(x, edge_index, edge_type, basis1, comp1, root1, bias1,
           basis2, comp2, root2, bias2):
    src = edge_index[0]
    dst = edge_index[1]

    gidx2, cidx2 = _tc_edge(
        src.reshape(2500, 128), dst.reshape(2500, 128),
        edge_type.reshape(2500, 128))
    gidx = gidx2.reshape(E)
    cidx = cidx2.reshape(E)

    counts = _sc_count(cidx)
    inv = _tc_inv(counts.reshape(NCORES, NR_PAD // 128, 128)).reshape(NR_PAD)
    norm = _sc_norm(cidx, inv)

    wflat1 = jnp.einsum("rb,bio->iro", comp1, basis1).reshape(IN_C, R * HID)
    wflat2 = jnp.einsum("rb,bio->iro", comp2, basis2).reshape(HID, R * NC)

    h1, xr1 = _tc_prep(x, wflat1, root1, bias1.reshape(1, 16))
    agg1 = _sc_msg(gidx, dst, norm, h1.reshape(NR, 16))
    agg1 = agg1.reshape(NCORES, N_PAD, 16)[:, :N, :]

    h2, hr2 = _tc_mid(agg1, xr1, wflat2, root2, bias2.reshape(1, 16))
    agg2 = _sc_msg(gidx, dst, norm, h2.reshape(NR, 16))
    agg2 = agg2.reshape(NCORES, N_PAD, 16)[:, :N, :]

    return _tc_fin(agg2, hr2)


# speedup vs baseline: 24.3988x; 3.0144x over previous
import functools

import jax
import jax.numpy as jnp
from jax import lax
from jax.experimental import pallas as pl
from jax.experimental.pallas import tpu as pltpu
from jax.experimental.pallas import tpu_sc as plsc

N = 10000
E = 320000
IN_C = 128
HID = 16
R = 40
NC = 16
NR = N * R
NR_PAD = 409600

NCORES = 2
NSUB = 16
NW = NCORES * NSUB
EW = E // NW
BT = 80
NBATCH = EW // BT

N_PAD = 10240
ROWS_T = N_PAD // NSUB
CNT_T = NR_PAD // NSUB
CNT_CH = 3200
NB_ROWS = 2000
GRID_N = N // NB_ROWS


def _sc_mesh():
    return plsc.VectorSubcoreMesh(core_axis_name="c", subcore_axis_name="s")


_SC_PARAMS = pltpu.CompilerParams(use_tc_tiling_on_sc=False)


def _fill1d(ref, n, val):
    @pl.loop(0, n // 16)
    def _(i):
        ref[pl.ds(i * 16, 16)] = jnp.full((16,), val, ref.dtype)


def _fill2d(ref, rows, val):
    @pl.loop(0, rows)
    def _(i):
        ref[i, :] = jnp.full((16,), val, ref.dtype)


@functools.partial(
    pl.kernel,
    out_type=jax.ShapeDtypeStruct((NCORES * NR_PAD,), jnp.float32),
    mesh=_sc_mesh(),
    compiler_params=_SC_PARAMS,
    scratch_types=[
        pltpu.VMEM((NBATCH, BT), jnp.int32),
        pltpu.VMEM((BT,), jnp.float32),
        pltpu.VMEM((CNT_CH,), jnp.float32),
        pltpu.VMEM_SHARED((NR_PAD,), jnp.float32),
    ],
)
def _sc_count(cidx_hbm, out_hbm, cidx_v, ones_v, buf_v, cnt_sh):
    c = lax.axis_index("c")
    s = lax.axis_index("s")
    wid = s * NCORES + c

    pltpu.sync_copy(cidx_hbm.at[wid], cidx_v)
    _fill1d(buf_v, CNT_CH, 0.0)

    @pl.loop(0, CNT_T // CNT_CH)
    def _(k):
        off = pl.multiple_of(s * CNT_T + k * CNT_CH, 8)
        pltpu.sync_copy(buf_v, cnt_sh.at[pl.ds(off, CNT_CH)])

    _fill1d(ones_v, BT, 1.0)
    plsc.subcore_barrier()

    @pl.loop(0, NBATCH)
    def _(j):
        pltpu.sync_copy(ones_v, cnt_sh.at[cidx_v.at[j]], add=True)

    plsc.subcore_barrier()

    @pl.loop(0, CNT_T // CNT_CH)
    def _(k):
        off = pl.multiple_of(s * CNT_T + k * CNT_CH, 8)
        pltpu.sync_copy(cnt_sh.at[pl.ds(off, CNT_CH)], buf_v)
        dst_off = pl.multiple_of(c * NR_PAD + s * CNT_T + k * CNT_CH, 8)
        pltpu.sync_copy(buf_v, out_hbm.at[pl.ds(dst_off, CNT_CH)])


def _msg_prologue(gidx_hbm, dst_hbm, gidx_v, dst_v, tile_v, agg_sh, wid, s):
    pltpu.sync_copy(gidx_hbm.at[wid], gidx_v)
    pltpu.sync_copy(dst_hbm.at[wid], dst_v)
    _fill2d(tile_v, ROWS_T, 0.0)
    row0 = pl.multiple_of(s * ROWS_T, 8)
    pltpu.sync_copy(tile_v, agg_sh.at[pl.ds(row0, ROWS_T), :])
    return row0


def _msg_scale_scatter(rv, norm_v, j, dst_v, agg_sh):
    @pl.loop(0, BT // 16)
    def _(g):
        nv = norm_v[j, pl.ds(g * 16, 16)]
        for k in range(16):
            e = g * 16 + k
            rv[e, :] = rv[e, :] * nv[k]

    pltpu.sync_copy(rv, agg_sh.at[dst_v.at[j]], add=True)


def _msg_epilogue(agg_sh, tile_v, agg_hbm, row0, c, s):
    plsc.subcore_barrier()
    pltpu.sync_copy(agg_sh.at[pl.ds(row0, ROWS_T), :], tile_v)
    out_row = pl.multiple_of(c * N_PAD + s * ROWS_T, 8)
    pltpu.sync_copy(tile_v, agg_hbm.at[pl.ds(out_row, ROWS_T), :])


_MSG_SCRATCH1 = [
    pltpu.VMEM((NBATCH, BT), jnp.int32),
    pltpu.VMEM((NBATCH, BT), jnp.int32),
    pltpu.VMEM((NBATCH, BT), jnp.int32),
    pltpu.VMEM((NBATCH, BT), jnp.float32),
    pltpu.VMEM((BT, 16), jnp.float32),
    pltpu.VMEM((BT, 16), jnp.float32),
    pltpu.VMEM((ROWS_T, 16), jnp.float32),
    pltpu.VMEM_SHARED((N_PAD, 16), jnp.float32),
    pltpu.SemaphoreType.DMA,
    pltpu.SemaphoreType.DMA,
    pltpu.SemaphoreType.DMA,
    pltpu.SemaphoreType.DMA,
]


@functools.partial(
    pl.kernel,
    out_type=(
        jax.ShapeDtypeStruct((NCORES * N_PAD, 16), jnp.float32),
        jax.ShapeDtypeStruct((NW, NBATCH, BT), jnp.float32),
    ),
    mesh=_sc_mesh(),
    compiler_params=_SC_PARAMS,
    scratch_types=_MSG_SCRATCH1,
)
def _sc_msg1(gidx_hbm, dst_hbm, cidx_hbm, inv_hbm, h_hbm,
             agg_hbm, norm_out_hbm,
             gidx_v, dst_v, cidx_v, norm_v, rows0, rows1, tile_v, agg_sh,
             gsem0, gsem1, nsem0, nsem1):
    c = lax.axis_index("c")
    s = lax.axis_index("s")
    wid = s * NCORES + c

    pltpu.sync_copy(cidx_hbm.at[wid], cidx_v)
    row0 = _msg_prologue(gidx_hbm, dst_hbm, gidx_v, dst_v, tile_v, agg_sh,
                         wid, s)
    plsc.subcore_barrier()

    rows = (rows0, rows1)
    gsems = (gsem0, gsem1)
    nsems = (nsem0, nsem1)

    def issue(j, b):
        pltpu.async_copy(h_hbm.at[gidx_v.at[j]], rows[b], gsems[b])
        pltpu.async_copy(inv_hbm.at[cidx_v.at[j]], norm_v.at[j], nsems[b])

    def process(j, b):
        pltpu.make_async_copy(
            h_hbm.at[pl.ds(0, BT), :], rows[b], gsems[b]).wait()
        pltpu.make_async_copy(
            inv_hbm.at[pl.ds(0, BT)], norm_v.at[j], nsems[b]).wait()
        _msg_scale_scatter(rows[b], norm_v, j, dst_v, agg_sh)

    issue(0, 0)

    @pl.loop(0, NBATCH - 1, step=2)
    def _(j):
        issue(j + 1, 1)
        process(j, 0)
        issue(j + 2, 0)
        process(j + 1, 1)

    process(NBATCH - 1, 0)

    pltpu.sync_copy(norm_v, norm_out_hbm.at[wid])
    _msg_epilogue(agg_sh, tile_v, agg_hbm, row0, c, s)


_MSG_SCRATCH2 = [
    pltpu.VMEM((NBATCH, BT), jnp.int32),
    pltpu.VMEM((NBATCH, BT), jnp.int32),
    pltpu.VMEM((NBATCH, BT), jnp.float32),
    pltpu.VMEM((BT, 16), jnp.float32),
    pltpu.VMEM((BT, 16), jnp.float32),
    pltpu.VMEM((ROWS_T, 16), jnp.float32),
    pltpu.VMEM_SHARED((N_PAD, 16), jnp.float32),
    pltpu.SemaphoreType.DMA,
    pltpu.SemaphoreType.DMA,
]


@functools.partial(
    pl.kernel,
    out_type=jax.ShapeDtypeStruct((NCORES * N_PAD, 16), jnp.float32),
    mesh=_sc_mesh(),
    compiler_params=_SC_PARAMS,
    scratch_types=_MSG_SCRATCH2,
)
def _sc_msg2(gidx_hbm, dst_hbm, norm_hbm, h_hbm, agg_hbm,
             gidx_v, dst_v, norm_v, rows0, rows1, tile_v, agg_sh,
             gsem0, gsem1):
    c = lax.axis_index("c")
    s = lax.axis_index("s")
    wid = s * NCORES + c

    pltpu.sync_copy(norm_hbm.at[wid], norm_v)
    row0 = _msg_prologue(gidx_hbm, dst_hbm, gidx_v, dst_v, tile_v, agg_sh,
                         wid, s)
    plsc.subcore_barrier()

    rows = (rows0, rows1)
    gsems = (gsem0, gsem1)

    def issue(j, b):
        pltpu.async_copy(h_hbm.at[gidx_v.at[j]], rows[b], gsems[b])

    def process(j, b):
        pltpu.make_async_copy(
            h_hbm.at[pl.ds(0, BT), :], rows[b], gsems[b]).wait()
        _msg_scale_scatter(rows[b], norm_v, j, dst_v, agg_sh)

    issue(0, 0)

    @pl.loop(0, NBATCH - 1, step=2)
    def _(j):
        issue(j + 1, 1)
        process(j, 0)
        issue(j + 2, 0)
        process(j + 1, 1)

    process(NBATCH - 1, 0)
    _msg_epilogue(agg_sh, tile_v, agg_hbm, row0, c, s)


def _edge_body(s_ref, d_ref, t_ref, g_ref, c_ref):
    t = t_ref[...]
    g_ref[...] = s_ref[...] * R + t
    c_ref[...] = d_ref[...] * R + t


def _tc_edge(src2, dst2, et2):
    return pl.pallas_call(
        _edge_body,
        out_shape=(
            jax.ShapeDtypeStruct(src2.shape, jnp.int32),
            jax.ShapeDtypeStruct(src2.shape, jnp.int32),
        ),
    )(src2, dst2, et2)


def _inv_body(c_ref, o_ref):
    o_ref[...] = 1.0 / jnp.maximum(c_ref[0] + c_ref[1], 1.0)


def _tc_inv(counts3):
    return pl.pallas_call(
        _inv_body,
        out_shape=jax.ShapeDtypeStruct(counts3.shape[1:], jnp.float32),
    )(counts3)


def _prep_body(x_ref, wf_ref, r_ref, b_ref, h_ref, xr_ref):
    xv = x_ref[...]
    h_ref[...] = jnp.dot(xv, wf_ref[...], preferred_element_type=jnp.float32)
    xr_ref[...] = (
        jnp.dot(xv, r_ref[...], preferred_element_type=jnp.float32) + b_ref[...]
    )


def _tc_prep(x, wflat, root, bias2d):
    k = x.shape[1]
    m = wflat.shape[1]
    return pl.pallas_call(
        _prep_body,
        grid=(GRID_N,),
        in_specs=[
            pl.BlockSpec((NB_ROWS, k), lambda i: (i, 0)),
            pl.BlockSpec((k, m), lambda i: (0, 0)),
            pl.BlockSpec((k, 16), lambda i: (0, 0)),
            pl.BlockSpec((1, 16), lambda i: (0, 0)),
        ],
        out_specs=(
            pl.BlockSpec((NB_ROWS, m), lambda i: (i, 0)),
            pl.BlockSpec((NB_ROWS, 16), lambda i: (i, 0)),
        ),
        out_shape=(
            jax.ShapeDtypeStruct((N, m), jnp.float32),
            jax.ShapeDtypeStruct((N, 16), jnp.float32),
        ),
    )(x, wflat, root, bias2d)


def _mid_body(agg_ref, xr_ref, wf_ref, r_ref, b_ref, h2_ref, hr_ref):
    h = jnp.maximum(agg_ref[0] + agg_ref[1] + xr_ref[...], 0.0)
    h2_ref[...] = jnp.dot(h, wf_ref[...], preferred_element_type=jnp.float32)
    hr_ref[...] = (
        jnp.dot(h, r_ref[...], preferred_element_type=jnp.float32) + b_ref[...]
    )


def _tc_mid(agg3, xr, wflat2, root2, bias2d):
    m = wflat2.shape[1]
    return pl.pallas_call(
        _mid_body,
        grid=(GRID_N,),
        in_specs=[
            pl.BlockSpec((2, NB_ROWS, 16), lambda i: (0, i, 0)),
            pl.BlockSpec((NB_ROWS, 16), lambda i: (i, 0)),
            pl.BlockSpec((16, m), lambda i: (0, 0)),
            pl.BlockSpec((16, 16), lambda i: (0, 0)),
            pl.BlockSpec((1, 16), lambda i: (0, 0)),
        ],
        out_specs=(
            pl.BlockSpec((NB_ROWS, m), lambda i: (i, 0)),
            pl.BlockSpec((NB_ROWS, 16), lambda i: (i, 0)),
        ),
        out_shape=(
            jax.ShapeDtypeStruct((N, m), jnp.float32),
            jax.ShapeDtypeStruct((N, 16), jnp.float32),
        ),
    )(agg3, xr, wflat2, root2, bias2d)


def _fin_body(agg_ref, hr_ref, o_ref):
    v = agg_ref[0] + agg_ref[1] + hr_ref[...]
    m = jnp.max(v, axis=1, keepdims=True)
    ex = jnp.exp(v - m)
    o_ref[...] = (v - m) - jnp.log(jnp.sum(ex, axis=1, keepdims=True))


def _tc_fin(agg3, hr):
    return pl.pallas_call(
        _fin_body,
        grid=(GRID_N,),
        in_specs=[
            pl.BlockSpec((2, NB_ROWS, 16), lambda i: (0, i, 0)),
            pl.BlockSpec((NB_ROWS, 16), lambda i: (i, 0)),
        ],
        out_specs=pl.BlockSpec((NB_ROWS, 16), lambda i: (i, 0)),
        out_shape=jax.ShapeDtypeStruct((N, 16), jnp.float32),
    )(agg3, hr)


def kernel(x, edge_index, edge_type, basis1, comp1, root1, bias1,
           basis2, comp2, root2, bias2):
    src = edge_index[0]
    dst = edge_index[1]

    gidx2, cidx2 = _tc_edge(
        src.reshape(2500, 128), dst.reshape(2500, 128),
        edge_type.reshape(2500, 128))
    gidx3 = gidx2.reshape(NW, NBATCH, BT)
    cidx3 = cidx2.reshape(NW, NBATCH, BT)
    dst3 = dst.reshape(NW, NBATCH, BT)

    counts = _sc_count(cidx3)
    inv = _tc_inv(counts.reshape(NCORES, NR_PAD // 128, 128)).reshape(NR_PAD)

    wflat1 = jnp.einsum("rb,bio->iro", comp1, basis1).reshape(IN_C, R * HID)
    wflat2 = jnp.einsum("rb,bio->iro", comp2, basis2).reshape(HID, R * NC)

    h1, xr1 = _tc_prep(x, wflat1, root1, bias1.reshape(1, 16))
    agg1, norm3 = _sc_msg1(gidx3, dst3, cidx3, inv, h1.reshape(NR, 16))
    agg1 = agg1.reshape(NCORES, N_PAD, 16)[:, :N, :]

    h2, hr2 = _tc_mid(agg1, xr1, wflat2, root2, bias2.reshape(1, 16))
    agg2 = _sc_msg2(gidx3, dst3, norm3, h2.reshape(NR, 16))
    agg2 = agg2.reshape(NCORES, N_PAD, 16)[:, :N, :]

    return _tc_fin(agg2, hr2)
